# baseline (speedup 1.0000x reference)
import functools

import jax
import jax.numpy as jnp
from jax.experimental import pallas as pl
from jax.experimental.pallas import tpu as pltpu


def _support_body(x_ref, w_ref, sup_ref):
    sup_ref[...] = jnp.dot(
        x_ref[...], w_ref[...], preferred_element_type=jnp.float32)


def _support(x, w):
    n = x.shape[0]
    f = w.shape[1]
    return pl.pallas_call(
        _support_body,
        out_shape=jax.ShapeDtypeStruct((n, f), jnp.float32),
    )(x, w)


def _fused_body(*refs, side_len, relu, avg, cast, emit, nxt, uw):
    it = iter(refs)
    adj_ref = next(it)
    u_ref = next(it)
    sup_ref = next(it)
    b_ref = next(it)
    wn_ref = next(it) if nxt else None
    res_ref = next(it) if avg else None
    adjout_ref = next(it) if cast else None
    out_ref = next(it) if emit else None
    supn_ref = next(it) if nxt else None
    ubf_ref = next(it)

    @pl.when(pl.program_id(0) == 0)
    def _():
        ubf_ref[...] = u_ref[...].astype(jnp.bfloat16)

    adj = adj_ref[...]
    if cast:
        adj = adj.astype(jnp.bfloat16)
        adjout_ref[...] = adj
    s1 = jnp.dot(adj, ubf_ref[...], preferred_element_type=jnp.float32)
    sup = sup_ref[...]
    z = jnp.concatenate([s1, sup[:, s1.shape[1]:]], axis=1)
    if side_len != s1.shape[1]:
        col = jax.lax.broadcasted_iota(jnp.int32, z.shape, 1)
        z = jnp.where(col < side_len, z, sup)
    y = z + b_ref[...]
    if relu:
        y = jnp.maximum(y, 0.0)
    if avg:
        y = (res_ref[...] + y) * 0.5
    if emit:
        out_ref[...] = y
    if nxt:
        supn_ref[...] = jnp.dot(
            y, wn_ref[...], preferred_element_type=jnp.float32)


def _fused_layer(adj_in, usrc, sup, b, wn, res, bm, side_len, relu, cast,
                 emit, uw):
    n, f = sup.shape
    nxt = wn is not None
    adj3 = adj_in.ndim == 3

    in_specs = [
        pl.BlockSpec((None, bm, n), lambda i: (i, 0, 0)) if adj3
        else pl.BlockSpec((bm, n), lambda i: (i, 0)),
        pl.BlockSpec((n, uw), lambda i: (0, 0)),
        pl.BlockSpec((bm, f), lambda i: (i, 0)),
        pl.BlockSpec((1, f), lambda i: (0, 0)),
    ]
    args = [adj_in, usrc, sup, jnp.reshape(b, (1, f))]
    if nxt:
        in_specs.append(pl.BlockSpec(wn.shape, lambda i: (0, 0)))
        args.append(wn)
    if res is not None:
        in_specs.append(pl.BlockSpec((bm, f), lambda i: (i, 0)))
        args.append(res)

    out_specs = []
    out_shape = []
    if cast:
        out_specs.append(pl.BlockSpec((bm, n), lambda i: (i, 0)))
        out_shape.append(jax.ShapeDtypeStruct((n, n), jnp.bfloat16))
    if emit:
        out_specs.append(pl.BlockSpec((bm, f), lambda i: (i, 0)))
        out_shape.append(jax.ShapeDtypeStruct((n, f), jnp.float32))
    if nxt:
        fn = wn.shape[1]
        out_specs.append(pl.BlockSpec((bm, fn), lambda i: (i, 0)))
        out_shape.append(jax.ShapeDtypeStruct((n, fn), jnp.float32))

    outs = pl.pallas_call(
        functools.partial(
            _fused_body, side_len=side_len, relu=relu, avg=res is not None,
            cast=cast, emit=emit, nxt=nxt, uw=uw,
        ),
        grid=(n // bm,),
        in_specs=in_specs,
        out_specs=tuple(out_specs),
        out_shape=tuple(out_shape),
        scratch_shapes=[pltpu.VMEM((n, uw), jnp.bfloat16)],
    )(*args)
    return list(outs) if isinstance(outs, (tuple, list)) else [outs]


def kernel(features, adj, Ws, bs):
    n = features.shape[0]
    h = Ws[0].shape[1]
    out_d = Ws[-1].shape[1]
    sl = max(h // 3, 2)
    bm0 = 400 if n % 400 == 0 else n
    bm = 1000 if n % 1000 == 0 else n

    w_last = jnp.pad(Ws[13], ((0, 0), (0, 128 - out_d)))
    b_last = jnp.pad(bs[13], ((0, 128 - out_d),))

    sup = _support(features, Ws[0])
    adj_bf, sup = _fused_layer(
        adj, sup[:, :sl], sup, bs[0], Ws[1], None, bm0, sl, True, True,
        False, sl)
    adj_v = jnp.reshape(adj_bf, (n // bm, bm, n))
    feats, sup = _fused_layer(
        adj_v, sup[:, :sl], sup, bs[1], Ws[2], features[:, :h], bm, sl, True,
        False, True, sl)
    for i in (2, 4, 6, 8, 10):
        (sup,) = _fused_layer(
            adj_v, sup[:, :sl], sup, bs[i], Ws[i + 1], None, bm, sl, True,
            False, False, sl)
        feats, sup = _fused_layer(
            adj_v, sup[:, :sl], sup, bs[i + 1], Ws[i + 2], feats, bm, sl,
            True, False, True, sl)
    feats, sup = _fused_layer(
        adj_v, sup[:, :sl], sup, bs[12], w_last, feats, bm, sl, True, False,
        True, sl)
    (coords_p,) = _fused_layer(
        adj_v, sup[:, :sl], sup, b_last, None, None, bm, max(out_d // 3, 2),
        False, False, True, sl)
    return coords_p[:, :out_d], feats
